# baseline (speedup 1.0000x reference)
import functools

import jax
import jax.numpy as jnp
from jax import lax
from jax.experimental import pallas as pl
from jax.experimental.pallas import tpu as pltpu


def _select_mask(acc, k):
    s = lax.bitcast_convert_type(acc, jnp.int32)

    count_pos = jnp.sum(jnp.where(s >= 0, jnp.int32(1), jnp.int32(0)))
    use_pos = count_pos >= k
    kk = jnp.where(use_pos, k, k - count_pos)

    key = jnp.where(use_pos,
                    jnp.where(s >= 0, s, -1),
                    jnp.where(s < 0, ~s, -1))

    def body(i, v):
        cand = v | (jnp.int32(1) << (30 - i))
        c = jnp.sum(jnp.where(key >= cand, jnp.int32(1), jnp.int32(0)))
        return jnp.where(c >= kk, cand, v)

    v = lax.fori_loop(0, 31, body, jnp.int32(0))

    pos_arm = jnp.where(s >= v, jnp.int32(1), jnp.int32(0))
    neg_arm = jnp.where((s >= 0) | (~s >= v), jnp.int32(1), jnp.int32(0))
    return jnp.where(use_pos, pos_arm, neg_arm)


def _fused_kernel(k, TT, x_ref, w_ref, lg_ref, mask_ref, acc_ref):
    t = pl.program_id(1)
    r = lax.dot_general(
        w_ref[...], x_ref[0], (((1,), (1,)), ((), ())),
        preferred_element_type=jnp.float32)
    row = r[0:1, :]
    lg_ref[0, 0, :, :] = row
    acc_ref[pl.ds(t, 1), :] = row

    @pl.when(t == TT - 1)
    def _():
        mask_ref[0, 0, :, :] = _select_mask(acc_ref[...], k)


def kernel(x, W):
    B, T, D = x.shape
    k = max(1, int(T * 0.5))
    Tt = 1024
    TT = T // Tt

    W8 = jnp.zeros((8, D), jnp.float32).at[0].set(W[0])

    logits4, mask4 = pl.pallas_call(
        functools.partial(_fused_kernel, k, TT),
        grid=(B, TT),
        in_specs=[
            pl.BlockSpec((1, Tt, D), lambda b, t: (b, t, 0)),
            pl.BlockSpec((8, D), lambda b, t: (0, 0)),
        ],
        out_specs=[
            pl.BlockSpec((1, 1, 1, Tt), lambda b, t: (b, t, 0, 0)),
            pl.BlockSpec((1, 1, TT, Tt), lambda b, t: (b, 0, 0, 0)),
        ],
        out_shape=[
            jax.ShapeDtypeStruct((B, TT, 1, Tt), jnp.float32),
            jax.ShapeDtypeStruct((B, 1, TT, Tt), jnp.int32),
        ],
        scratch_shapes=[pltpu.VMEM((TT, Tt), jnp.float32)],
    )(x, W8)

    return (mask4.reshape(B, T).astype(jnp.bool_), logits4.reshape(B, T))


# speedup vs baseline: 1.5485x; 1.5485x over previous
import functools

import jax
import jax.numpy as jnp
from jax import lax
from jax.experimental import pallas as pl
from jax.experimental.pallas import tpu as pltpu


def _count(cond):
    return jnp.sum(jnp.where(cond, jnp.int32(1), jnp.int32(0)),
                   axis=(1, 2), keepdims=True)


def _select_mask(acc, k):
    s = lax.bitcast_convert_type(acc, jnp.int32)

    count_pos = _count(s >= 0)
    use_pos = count_pos >= k
    kk = jnp.where(use_pos, jnp.int32(k), k - count_pos)

    key = jnp.where(use_pos,
                    jnp.where(s >= 0, s, -1),
                    jnp.where(s < 0, ~s, -1))

    v = jnp.zeros(kk.shape, jnp.int32)
    cand = v | jnp.int32(1 << 30)
    v = jnp.where(_count(key >= cand) >= kk, cand, v)
    for sh in range(27, -1, -3):
        digit = jnp.zeros(kk.shape, jnp.int32)
        for m in range(1, 8):
            c_m = _count(key >= (v | jnp.int32(m << sh)))
            digit = digit + jnp.where(c_m >= kk, jnp.int32(1), jnp.int32(0))
        v = v | jnp.left_shift(digit, sh)

    pos_arm = jnp.where(s >= v, jnp.int32(1), jnp.int32(0))
    neg_arm = jnp.where((s >= 0) | (~s >= v), jnp.int32(1), jnp.int32(0))
    return jnp.where(use_pos, pos_arm, neg_arm)


def _fused_kernel(k, B, TT, C, x_ref, w_ref, lg_ref, mask_ref, acc_ref):
    b = pl.program_id(0)
    t = pl.program_id(1)
    r = lax.dot_general(
        w_ref[...], x_ref[0], (((1,), (1,)), ((), ())),
        preferred_element_type=jnp.float32)
    row = r[0:1, :]
    lg_ref[0, 0, :, :] = row
    J = row.shape[1] // C
    for j in range(J):
        acc_ref[pl.ds(b, 1), pl.ds(t * J + j, 1), :] = (
            row[:, j * C:(j + 1) * C][None])

    @pl.when((b == B - 1) & (t == TT - 1))
    def _():
        mask_ref[...] = _select_mask(acc_ref[...], k)


def kernel(x, W):
    B, T, D = x.shape
    k = max(1, int(T * 0.5))
    Tt = 4096
    TT = T // Tt
    C = 1024

    W8 = jnp.zeros((8, D), jnp.float32).at[0].set(W[0])

    logits4, mask3 = pl.pallas_call(
        functools.partial(_fused_kernel, k, B, TT, C),
        grid=(B, TT),
        in_specs=[
            pl.BlockSpec((1, Tt, D), lambda b, t: (b, t, 0)),
            pl.BlockSpec((8, D), lambda b, t: (0, 0)),
        ],
        out_specs=[
            pl.BlockSpec((1, 1, 1, Tt), lambda b, t: (b, t, 0, 0)),
            pl.BlockSpec((B, T // C, C), lambda b, t: (0, 0, 0)),
        ],
        out_shape=[
            jax.ShapeDtypeStruct((B, TT, 1, Tt), jnp.float32),
            jax.ShapeDtypeStruct((B, T // C, C), jnp.int32),
        ],
        scratch_shapes=[pltpu.VMEM((B, T // C, C), jnp.float32)],
    )(x, W8)

    return (mask3.reshape(B, T).astype(jnp.bool_), logits4.reshape(B, T))
